# baseline (speedup 1.0000x reference)
import functools

import numpy as np
import jax
import jax.numpy as jnp
from jax import lax
from jax.experimental import pallas as pl
from jax.experimental.pallas import tpu as pltpu
from jax.experimental.pallas import tpu_sc as plsc

_D = 16
_LANES = 128


def _pos_encoding(seq_len, embed_dim):
    pos = np.arange(seq_len)[:, np.newaxis]
    i = np.arange(embed_dim)[np.newaxis, :]
    angle_rates = 1.0 / np.power(10000, 2 * (i // 2) / np.float32(embed_dim))
    a = pos * angle_rates
    a[:, 0::2] = np.sin(a[:, 0::2])
    a[:, 1::2] = np.cos(a[:, 1::2])
    return a.astype(np.float32)


@functools.lru_cache(maxsize=None)
def _build(seq_len, batch, vocab):
    nc, ns = 2, 16
    nw = nc * ns
    assert batch == nw * _LANES and seq_len % 2 == 0
    n_dg = _D // 8

    mesh = plsc.VectorSubcoreMesh(core_axis_name="c", subcore_axis_name="s")

    nb = 8
    assert seq_len % nb == 0 and seq_len >= 2 * nb

    @functools.partial(
        pl.kernel,
        out_type=jax.ShapeDtypeStruct((seq_len, n_dg, nw, 8 * _LANES),
                                      jnp.float32),
        mesh=mesh,
        scratch_types=[
            pltpu.VMEM((seq_len, _LANES), jnp.int32),
            pltpu.VMEM((nb, _LANES, _D), jnp.float32),
            pltpu.VMEM((nb, n_dg * 8 * _LANES), jnp.float32),
            pltpu.VMEM((seq_len, _D), jnp.float32),
            [pltpu.SemaphoreType.DMA] * nb,
            [pltpu.SemaphoreType.DMA] * nb,
        ],
        compiler_params=pltpu.CompilerParams(
            use_tc_tiling_on_sc=False, needs_layout_passes=False),
    )
    def run(x_hbm, pos_hbm, table_hbm, out_hbm, xblk, gbuf, obuf, pos_v,
            gsems, osems):
        w = lax.axis_index("s") * nc + lax.axis_index("c")
        col0 = w * _LANES
        pltpu.sync_copy(pos_hbm, pos_v)
        pltpu.sync_copy(x_hbm.at[:, pl.ds(col0, _LANES)], xblk)

        def gather_copy(s, t):
            return pltpu.async_copy(
                table_hbm.at[xblk.at[s]], gbuf.at[t], gsems[t])

        def gather_wait(s, t):
            pltpu.make_async_copy(
                table_hbm.at[xblk.at[s]], gbuf.at[t], gsems[t]).wait()

        def out_copy(s, t, dg):
            return pltpu.async_copy(
                obuf.at[t].at[pl.ds(dg * 8 * _LANES, 8 * _LANES)],
                out_hbm.at[s, dg, w], osems[t])

        def out_wait(s, t, dg):
            pltpu.make_async_copy(
                obuf.at[t].at[pl.ds(dg * 8 * _LANES, 8 * _LANES)],
                out_hbm.at[s, dg, w], osems[t]).wait()

        iota16 = lax.iota(jnp.int32, 16)
        sidx = (iota16 // 8) * (8 * _LANES) + (iota16 % 8) * _LANES

        def compute(s, t):
            g = gbuf.at[t]
            ofl = obuf.at[t]
            pv = pos_v[s]

            @plsc.parallel_loop(0, _LANES, unroll=8)
            def vloop(i):
                vals = g[i] + pv
                plsc.store_scatter(ofl, [sidx + i], vals)

        for t in range(nb - 1):
            gather_copy(t, t)

        def chunk_body(s8, carry):
            s_base = s8 * nb
            for t in range(nb):
                s = s_base + t
                gather_wait(s, t)
                tg = (t - 1) % nb
                @pl.when(s + nb - 1 < seq_len)
                def _():
                    gather_copy(s + nb - 1, tg)
                @pl.when(s >= nb)
                def _():
                    for dg in range(n_dg):
                        out_wait(s - nb, t, dg)
                compute(s, t)
                for dg in range(n_dg):
                    out_copy(s, t, dg)
            return carry

        lax.fori_loop(0, seq_len // nb, chunk_body, 0)
        for t in range(nb):
            for dg in range(n_dg):
                out_wait(seq_len - nb + t, t, dg)

    return run


def kernel(x_in, table):
    b, s = x_in.shape
    vocab, d = table.shape
    x_t = x_in.T.astype(jnp.int32)
    pos = jnp.asarray(_pos_encoding(s, d))
    out_lin = _build(s, b, vocab)(x_t, pos, table)
    nw = out_lin.shape[2]
    out5 = out_lin.reshape(s, d // 8, nw, 8, _LANES)
    return out5.transpose((2, 4, 0, 1, 3)).reshape(nw * _LANES, s, d)
